# Initial kernel scaffold; baseline (speedup 1.0000x reference)
import jax
import jax.numpy as jnp
from jax.experimental import pallas as pl


def kernel(x, edge_index, W1, b1, W2, b2, bn_gamma, bn_beta):
    raise NotImplementedError("write your pallas kernel here")


# speedup vs baseline: 3.9013x; 3.9013x over previous
import functools

import jax
import jax.numpy as jnp
from jax import lax
from jax.experimental import pallas as pl
from jax.experimental.pallas import tpu as pltpu
from jax.experimental.pallas import tpu_sc as plsc

N = 10000
E = 320000
D = 128
DH = D // 2

CH = 128
NTEC = 16
CHUNKS_PER_T = (E // CH + NTEC - 1) // NTEC
NCHUNK_PAD = CHUNKS_PER_T * NTEC
EP = NCHUNK_PAD * CH
N_PAD = 10240
ROWS_PER_TILE = N_PAD // NTEC


def _sc_aggregate():
    mesh = plsc.VectorSubcoreMesh(core_axis_name="c", subcore_axis_name="s")

    @functools.partial(
        pl.kernel,
        out_type=jax.ShapeDtypeStruct((2, N, DH), jnp.float32),
        mesh=mesh,
        compiler_params=pltpu.CompilerParams(use_tc_tiling_on_sc=False),
        scratch_types=[
            pltpu.VMEM((CH,), jnp.int32),
            pltpu.VMEM((CH,), jnp.int32),
            pltpu.VMEM((CH, DH), jnp.float32),
            pltpu.VMEM((ROWS_PER_TILE, DH), jnp.float32),
            pltpu.VMEM_SHARED((N_PAD, DH), jnp.float32),
            pltpu.SemaphoreType.DMA,
        ],
    )
    def sc_agg(xcat_hbm, srcp_hbm, dstp_hbm, zeros_hbm, out_hbm,
               src_v, dst_v, rows_v, zero_v, agg_sh, sem):
        c = lax.axis_index("c")
        s = lax.axis_index("s")
        r0 = s * ROWS_PER_TILE

        pltpu.sync_copy(zeros_hbm, zero_v)
        pltpu.sync_copy(zero_v, agg_sh.at[pl.ds(r0, ROWS_PER_TILE)])
        plsc.subcore_barrier()

        def body(t, carry):
            chunk = s * CHUNKS_PER_T + t
            pltpu.sync_copy(srcp_hbm.at[c, chunk], src_v)
            pltpu.sync_copy(dstp_hbm.at[chunk], dst_v)
            pltpu.async_copy(xcat_hbm.at[src_v], rows_v, sem).wait()
            pltpu.sync_copy(rows_v, agg_sh.at[dst_v], add=True)
            return carry

        lax.fori_loop(0, CHUNKS_PER_T, body, 0)
        plsc.subcore_barrier()

        @pl.when(r0 + ROWS_PER_TILE <= N)
        def _full():
            pltpu.sync_copy(agg_sh.at[pl.ds(r0, ROWS_PER_TILE)],
                            out_hbm.at[c, pl.ds(r0, ROWS_PER_TILE)])

        @pl.when(jnp.logical_and(r0 < N, r0 + ROWS_PER_TILE > N))
        def _tail():
            rb = (N // ROWS_PER_TILE) * ROWS_PER_TILE
            rem = N - rb
            pltpu.sync_copy(agg_sh.at[pl.ds(rb, rem)],
                            out_hbm.at[c, pl.ds(rb, rem)])

    return sc_agg


def _tc_body(x_ref, agg_ref, w1_ref, b1_ref, w2_ref, b2_ref, g_ref, be_ref,
             out_ref):
    agg = jnp.concatenate([agg_ref[0], agg_ref[1]], axis=-1)
    h = x_ref[...] + agg
    h1 = jnp.dot(h, w1_ref[...], preferred_element_type=jnp.float32)
    h1 = jnp.maximum(h1 + b1_ref[...], 0.0)
    h2 = jnp.dot(h1, w2_ref[...], preferred_element_type=jnp.float32)
    h2 = h2 + b2_ref[...]
    mean = jnp.mean(h2, axis=0, keepdims=True)
    var = jnp.mean(jnp.square(h2 - mean), axis=0, keepdims=True)
    hn = (h2 - mean) * lax.rsqrt(var + 1e-5) * g_ref[...] + be_ref[...]
    out_ref[...] = jnp.maximum(hn, 0.0)


@jax.jit
def kernel(x, edge_index, W1, b1, W2, b2, bn_gamma, bn_beta):
    src = edge_index[0]
    dst = edge_index[1]
    pad = EP - E
    srcf = jnp.concatenate([src, jnp.full((pad,), N_PAD - 1, jnp.int32)])
    srcp = jnp.stack([srcf, srcf + N_PAD]).reshape(2, NCHUNK_PAD, CH)
    dstp = jnp.concatenate(
        [dst, jnp.full((pad,), N_PAD - 1, jnp.int32)]).reshape(NCHUNK_PAD, CH)
    xp = jnp.zeros((N_PAD, D), jnp.float32).at[:N].set(x)
    xcat = jnp.concatenate([xp[:, :DH], xp[:, DH:]], axis=0)
    zeros = jnp.zeros((ROWS_PER_TILE, DH), jnp.float32)

    agg2 = _sc_aggregate()(xcat, srcp, dstp, zeros)

    out = pl.pallas_call(
        _tc_body,
        out_shape=jax.ShapeDtypeStruct((N, D), jnp.float32),
    )(x, agg2, W1, b1.reshape(1, D), W2, b2.reshape(1, D),
      bn_gamma.reshape(1, D), bn_beta.reshape(1, D))
    return out


# speedup vs baseline: 4.1193x; 1.0559x over previous
import functools

import jax
import jax.numpy as jnp
from jax import lax
from jax.experimental import pallas as pl
from jax.experimental.pallas import tpu as pltpu
from jax.experimental.pallas import tpu_sc as plsc

N = 10000
E = 320000
D = 128
DH = D // 2

CH = 128
NB = 4
NTEC = 16
BLOCKS_PER_T = 40
CHUNKS_PER_T = BLOCKS_PER_T * NB
NCHUNK_PAD = CHUNKS_PER_T * NTEC
NBLK = NCHUNK_PAD // NB
EP = NCHUNK_PAD * CH
N_PAD = 10240
ROWS_PER_TILE = N_PAD // NTEC


def _sc_aggregate():
    mesh = plsc.VectorSubcoreMesh(core_axis_name="c", subcore_axis_name="s")

    @functools.partial(
        pl.kernel,
        out_type=jax.ShapeDtypeStruct((2, N, DH), jnp.float32),
        mesh=mesh,
        compiler_params=pltpu.CompilerParams(use_tc_tiling_on_sc=False),
        scratch_types=[
            pltpu.VMEM((2, 2, NB, CH), jnp.int32),
            pltpu.VMEM((2, NB, CH, DH), jnp.float32),
            pltpu.VMEM_SHARED((N_PAD, DH), jnp.float32),
            pltpu.SemaphoreType.DMA,
            pltpu.SemaphoreType.DMA,
            pltpu.SemaphoreType.DMA,
            pltpu.SemaphoreType.DMA,
        ],
    )
    def sc_agg(xcat_hbm, eidx_hbm, zeros_hbm, out_hbm,
               idx_v, rows_v, agg_sh,
               gsem0, gsem1, ssem0, ssem1):
        c = lax.axis_index("c")
        s = lax.axis_index("s")
        r0 = s * ROWS_PER_TILE
        blk0 = s * BLOCKS_PER_T
        gsem = (gsem0, gsem1)
        ssem = (ssem0, ssem1)

        def fire_gathers(p, blk):
            for j in range(NB):
                pltpu.async_copy(xcat_hbm.at[idx_v.at[p, 0, j]],
                                 rows_v.at[p, j], gsem[p])

        def drain_gathers(p):
            for j in range(NB):
                pltpu.make_async_copy(xcat_hbm.at[idx_v.at[p, 0, j]],
                                      rows_v.at[p, j], gsem[p]).wait()

        def fire_scatters(p):
            for j in range(NB):
                pltpu.async_copy(rows_v.at[p, j],
                                 agg_sh.at[idx_v.at[p, 1, j]], ssem[p],
                                 add=True)

        def drain_scatters(p):
            for j in range(NB):
                pltpu.make_async_copy(rows_v.at[p, j],
                                      agg_sh.at[idx_v.at[p, 1, j]],
                                      ssem[p]).wait()

        pltpu.sync_copy(zeros_hbm, agg_sh.at[pl.ds(r0, ROWS_PER_TILE)])
        plsc.subcore_barrier()

        pltpu.sync_copy(eidx_hbm.at[c, blk0], idx_v.at[0])
        fire_gathers(0, blk0)

        def body(gi, carry):
            drain_gathers(0)

            @pl.when(gi > 0)
            def _():
                drain_scatters(1)

            pltpu.sync_copy(eidx_hbm.at[c, blk0 + 2 * gi + 1], idx_v.at[1])
            fire_gathers(1, blk0 + 2 * gi + 1)
            fire_scatters(0)

            drain_gathers(1)
            drain_scatters(0)

            @pl.when(gi < BLOCKS_PER_T // 2 - 1)
            def _():
                pltpu.sync_copy(eidx_hbm.at[c, blk0 + 2 * gi + 2], idx_v.at[0])
                fire_gathers(0, blk0 + 2 * gi + 2)

            fire_scatters(1)
            return carry

        lax.fori_loop(0, BLOCKS_PER_T // 2, body, 0)
        drain_scatters(1)
        plsc.subcore_barrier()

        @pl.when(r0 + ROWS_PER_TILE <= N)
        def _full():
            pltpu.sync_copy(agg_sh.at[pl.ds(r0, ROWS_PER_TILE)],
                            out_hbm.at[c, pl.ds(r0, ROWS_PER_TILE)])

        @pl.when(jnp.logical_and(r0 < N, r0 + ROWS_PER_TILE > N))
        def _tail():
            rb = (N // ROWS_PER_TILE) * ROWS_PER_TILE
            rem = N - rb
            pltpu.sync_copy(agg_sh.at[pl.ds(rb, rem)],
                            out_hbm.at[c, pl.ds(rb, rem)])

    return sc_agg


def _tc_body(x_ref, agg_ref, w1_ref, b1_ref, w2_ref, b2_ref, g_ref, be_ref,
             out_ref):
    agg = jnp.concatenate([agg_ref[0], agg_ref[1]], axis=-1)
    h = x_ref[...] + agg
    h1 = jnp.dot(h, w1_ref[...], preferred_element_type=jnp.float32)
    h1 = jnp.maximum(h1 + b1_ref[...], 0.0)
    h2 = jnp.dot(h1, w2_ref[...], preferred_element_type=jnp.float32)
    h2 = h2 + b2_ref[...]
    mean = jnp.mean(h2, axis=0, keepdims=True)
    var = jnp.mean(jnp.square(h2 - mean), axis=0, keepdims=True)
    hn = (h2 - mean) * lax.rsqrt(var + 1e-5) * g_ref[...] + be_ref[...]
    out_ref[...] = jnp.maximum(hn, 0.0)


@jax.jit
def kernel(x, edge_index, W1, b1, W2, b2, bn_gamma, bn_beta):
    src = edge_index[0]
    dst = edge_index[1]
    pad = EP - E
    srcf = jnp.concatenate([src, jnp.full((pad,), N_PAD - 1, jnp.int32)])
    dstf = jnp.concatenate([dst, jnp.full((pad,), N_PAD - 1, jnp.int32)])
    eidx = jnp.stack([
        jnp.stack([(srcf + c * N_PAD).reshape(NBLK, NB, CH),
                   dstf.reshape(NBLK, NB, CH)], axis=1)
        for c in (0, 1)
    ])
    xp = jnp.zeros((N_PAD, D), jnp.float32).at[:N].set(x)
    xcat = jnp.concatenate([xp[:, :DH], xp[:, DH:]], axis=0)
    zeros = jnp.zeros((ROWS_PER_TILE, DH), jnp.float32)

    agg2 = _sc_aggregate()(xcat, eidx, zeros)

    out = pl.pallas_call(
        _tc_body,
        out_shape=jax.ShapeDtypeStruct((N, D), jnp.float32),
    )(x, agg2, W1, b1.reshape(1, D), W2, b2.reshape(1, D),
      bn_gamma.reshape(1, D), bn_beta.reshape(1, D))
    return out


# speedup vs baseline: 6.7395x; 1.6361x over previous
import functools

import jax
import jax.numpy as jnp
from jax import lax
from jax.experimental import pallas as pl
from jax.experimental.pallas import tpu as pltpu
from jax.experimental.pallas import tpu_sc as plsc

N = 10000
E = 320000
D = 128
DH = D // 2

CH = 128
NB = 2
NTEC = 16
BLOCKS_PER_T = 80
CHUNKS_PER_T = BLOCKS_PER_T * NB
NCHUNK_PAD = CHUNKS_PER_T * NTEC
NBLK = NCHUNK_PAD // NB
EP = NCHUNK_PAD * CH
N_PAD = 10112
ROWS_PER_TILE = N_PAD // NTEC


def _sc_aggregate():
    mesh = plsc.VectorSubcoreMesh(core_axis_name="c", subcore_axis_name="s")

    @functools.partial(
        pl.kernel,
        out_type=jax.ShapeDtypeStruct((2, N, DH), jnp.float32),
        mesh=mesh,
        compiler_params=pltpu.CompilerParams(use_tc_tiling_on_sc=False),
        scratch_types=[
            pltpu.VMEM((2, 2, NB, CH), jnp.int32),
            pltpu.VMEM((2, NB, CH, DH), jnp.float32),
            pltpu.VMEM_SHARED((N_PAD, DH), jnp.float32),
            pltpu.VMEM_SHARED((N_PAD, DH), jnp.float32),
            pltpu.SemaphoreType.DMA,
            pltpu.SemaphoreType.DMA,
            pltpu.SemaphoreType.DMA,
            pltpu.SemaphoreType.DMA,
        ],
    )
    def sc_agg(xcat_hbm, eidx_hbm, zeros_hbm, out_hbm,
               idx_v, rows_v, agg_sh, x_sh,
               gsem0, gsem1, ssem0, ssem1):
        c = lax.axis_index("c")
        s = lax.axis_index("s")
        r0 = s * ROWS_PER_TILE
        blk0 = s * BLOCKS_PER_T
        gsem = (gsem0, gsem1)
        ssem = (ssem0, ssem1)

        def fire_gathers(p, blk):
            for j in range(NB):
                pltpu.async_copy(x_sh.at[idx_v.at[p, 0, j]],
                                 rows_v.at[p, j], gsem[p])

        def drain_gathers(p):
            for j in range(NB):
                pltpu.make_async_copy(x_sh.at[idx_v.at[p, 0, j]],
                                      rows_v.at[p, j], gsem[p]).wait()

        def fire_scatters(p):
            for j in range(NB):
                pltpu.async_copy(rows_v.at[p, j],
                                 agg_sh.at[idx_v.at[p, 1, j]], ssem[p],
                                 add=True)

        def drain_scatters(p):
            for j in range(NB):
                pltpu.make_async_copy(rows_v.at[p, j],
                                      agg_sh.at[idx_v.at[p, 1, j]],
                                      ssem[p]).wait()

        pltpu.sync_copy(zeros_hbm, agg_sh.at[pl.ds(r0, ROWS_PER_TILE)])
        pltpu.sync_copy(xcat_hbm.at[pl.ds(c * N_PAD + r0, ROWS_PER_TILE)],
                        x_sh.at[pl.ds(r0, ROWS_PER_TILE)])
        plsc.subcore_barrier()

        pltpu.sync_copy(eidx_hbm.at[blk0], idx_v.at[0])
        fire_gathers(0, blk0)

        def body(gi, carry):
            drain_gathers(0)

            @pl.when(gi > 0)
            def _():
                drain_scatters(1)

            pltpu.sync_copy(eidx_hbm.at[blk0 + 2 * gi + 1], idx_v.at[1])
            fire_gathers(1, blk0 + 2 * gi + 1)
            fire_scatters(0)

            drain_gathers(1)
            drain_scatters(0)

            @pl.when(gi < BLOCKS_PER_T // 2 - 1)
            def _():
                pltpu.sync_copy(eidx_hbm.at[blk0 + 2 * gi + 2], idx_v.at[0])
                fire_gathers(0, blk0 + 2 * gi + 2)

            fire_scatters(1)
            return carry

        lax.fori_loop(0, BLOCKS_PER_T // 2, body, 0)
        drain_scatters(1)
        plsc.subcore_barrier()

        @pl.when(r0 + ROWS_PER_TILE <= N)
        def _full():
            pltpu.sync_copy(agg_sh.at[pl.ds(r0, ROWS_PER_TILE)],
                            out_hbm.at[c, pl.ds(r0, ROWS_PER_TILE)])

        @pl.when(jnp.logical_and(r0 < N, r0 + ROWS_PER_TILE > N))
        def _tail():
            rb = (N // ROWS_PER_TILE) * ROWS_PER_TILE
            rem = N - rb
            pltpu.sync_copy(agg_sh.at[pl.ds(rb, rem)],
                            out_hbm.at[c, pl.ds(rb, rem)])

    return sc_agg


def _tc_body(x_ref, agg_ref, w1_ref, b1_ref, w2_ref, b2_ref, g_ref, be_ref,
             out_ref):
    agg = jnp.concatenate([agg_ref[0], agg_ref[1]], axis=-1)
    h = x_ref[...] + agg
    h1 = jnp.dot(h, w1_ref[...], preferred_element_type=jnp.float32)
    h1 = jnp.maximum(h1 + b1_ref[...], 0.0)
    h2 = jnp.dot(h1, w2_ref[...], preferred_element_type=jnp.float32)
    h2 = h2 + b2_ref[...]
    mean = jnp.mean(h2, axis=0, keepdims=True)
    var = jnp.mean(jnp.square(h2 - mean), axis=0, keepdims=True)
    hn = (h2 - mean) * lax.rsqrt(var + 1e-5) * g_ref[...] + be_ref[...]
    out_ref[...] = jnp.maximum(hn, 0.0)


@jax.jit
def kernel(x, edge_index, W1, b1, W2, b2, bn_gamma, bn_beta):
    src = edge_index[0]
    dst = edge_index[1]
    pad = EP - E
    srcf = jnp.concatenate([src, jnp.full((pad,), N_PAD - 1, jnp.int32)])
    dstf = jnp.concatenate([dst, jnp.full((pad,), N_PAD - 1, jnp.int32)])
    eidx = jnp.stack([srcf.reshape(NBLK, NB, CH),
                      dstf.reshape(NBLK, NB, CH)], axis=1)
    xp = jnp.zeros((N_PAD, D), jnp.float32).at[:N].set(x)
    xcat = jnp.concatenate([xp[:, :DH], xp[:, DH:]], axis=0)
    zeros = jnp.zeros((ROWS_PER_TILE, DH), jnp.float32)

    agg2 = _sc_aggregate()(xcat, eidx, zeros)

    out = pl.pallas_call(
        _tc_body,
        out_shape=jax.ShapeDtypeStruct((N, D), jnp.float32),
    )(x, agg2, W1, b1.reshape(1, D), W2, b2.reshape(1, D),
      bn_gamma.reshape(1, D), bn_beta.reshape(1, D))
    return out


# speedup vs baseline: 7.3154x; 1.0854x over previous
import functools

import jax
import jax.numpy as jnp
from jax import lax
from jax.experimental import pallas as pl
from jax.experimental.pallas import tpu as pltpu
from jax.experimental.pallas import tpu_sc as plsc

N = 10000
E = 320000
D = 128
DH = D // 2

CH = 128
NB = 2
NTEC = 16
BLOCKS_PER_T = 80
CHUNKS_PER_T = BLOCKS_PER_T * NB
NCHUNK_PAD = CHUNKS_PER_T * NTEC
NBLK = NCHUNK_PAD // NB
EP = NCHUNK_PAD * CH
N_PAD = 10112
ROWS_PER_TILE = N_PAD // NTEC


def _sc_aggregate():
    mesh = plsc.VectorSubcoreMesh(core_axis_name="c", subcore_axis_name="s")

    @functools.partial(
        pl.kernel,
        out_type=jax.ShapeDtypeStruct((2, N, DH), jnp.float32),
        mesh=mesh,
        compiler_params=pltpu.CompilerParams(use_tc_tiling_on_sc=False),
        scratch_types=[
            pltpu.VMEM((2, 2, NB, CH), jnp.int32),
            pltpu.VMEM((2, NB, CH, DH), jnp.float32),
            pltpu.VMEM_SHARED((N_PAD, DH), jnp.float32),
            pltpu.VMEM_SHARED((N_PAD, DH), jnp.float32),
            pltpu.SemaphoreType.DMA,
            pltpu.SemaphoreType.DMA,
            pltpu.SemaphoreType.DMA,
            pltpu.SemaphoreType.DMA,
        ],
    )
    def sc_agg(x_hbm, eidx_hbm, zeros_hbm, out_hbm,
               idx_v, rows_v, agg_sh, x_sh,
               gsem0, gsem1, ssem0, ssem1):
        c = lax.axis_index("c")
        s = lax.axis_index("s")
        r0 = s * ROWS_PER_TILE
        blk0 = s * BLOCKS_PER_T
        gsem = (gsem0, gsem1)
        ssem = (ssem0, ssem1)

        def fire_gathers(p, blk):
            for j in range(NB):
                pltpu.async_copy(x_sh.at[idx_v.at[p, 0, j]],
                                 rows_v.at[p, j], gsem[p])

        def drain_gathers(p):
            for j in range(NB):
                pltpu.make_async_copy(x_sh.at[idx_v.at[p, 0, j]],
                                      rows_v.at[p, j], gsem[p]).wait()

        def fire_scatters(p):
            for j in range(NB):
                pltpu.async_copy(rows_v.at[p, j],
                                 agg_sh.at[idx_v.at[p, 1, j]], ssem[p],
                                 add=True)

        def drain_scatters(p):
            for j in range(NB):
                pltpu.make_async_copy(rows_v.at[p, j],
                                      agg_sh.at[idx_v.at[p, 1, j]],
                                      ssem[p]).wait()

        pltpu.sync_copy(zeros_hbm, agg_sh.at[pl.ds(r0, ROWS_PER_TILE)])

        @pl.when(s < NTEC - 1)
        def _stage_full():
            pltpu.sync_copy(x_hbm.at[pl.ds(r0, ROWS_PER_TILE),
                                     pl.ds(c * DH, DH)],
                            x_sh.at[pl.ds(r0, ROWS_PER_TILE)])

        @pl.when(s == NTEC - 1)
        def _stage_tail():
            real = N - (NTEC - 1) * ROWS_PER_TILE
            pltpu.sync_copy(x_hbm.at[pl.ds((NTEC - 1) * ROWS_PER_TILE, real),
                                     pl.ds(c * DH, DH)],
                            x_sh.at[pl.ds((NTEC - 1) * ROWS_PER_TILE, real)])
            pltpu.sync_copy(zeros_hbm.at[pl.ds(0, N_PAD - N)],
                            x_sh.at[pl.ds(N, N_PAD - N)])

        plsc.subcore_barrier()

        pltpu.sync_copy(eidx_hbm.at[blk0], idx_v.at[0])
        fire_gathers(0, blk0)

        def body(gi, carry):
            drain_gathers(0)

            @pl.when(gi > 0)
            def _():
                drain_scatters(1)

            pltpu.sync_copy(eidx_hbm.at[blk0 + 2 * gi + 1], idx_v.at[1])
            fire_gathers(1, blk0 + 2 * gi + 1)
            fire_scatters(0)

            drain_gathers(1)
            drain_scatters(0)

            @pl.when(gi < BLOCKS_PER_T // 2 - 1)
            def _():
                pltpu.sync_copy(eidx_hbm.at[blk0 + 2 * gi + 2], idx_v.at[0])
                fire_gathers(0, blk0 + 2 * gi + 2)

            fire_scatters(1)
            return carry

        lax.fori_loop(0, BLOCKS_PER_T // 2, body, 0)
        drain_scatters(1)
        plsc.subcore_barrier()

        @pl.when(r0 + ROWS_PER_TILE <= N)
        def _full():
            pltpu.sync_copy(agg_sh.at[pl.ds(r0, ROWS_PER_TILE)],
                            out_hbm.at[c, pl.ds(r0, ROWS_PER_TILE)])

        @pl.when(jnp.logical_and(r0 < N, r0 + ROWS_PER_TILE > N))
        def _tail():
            rb = (N // ROWS_PER_TILE) * ROWS_PER_TILE
            rem = N - rb
            pltpu.sync_copy(agg_sh.at[pl.ds(rb, rem)],
                            out_hbm.at[c, pl.ds(rb, rem)])

    return sc_agg


def _tc_body(x_ref, agg_ref, w1_ref, b1_ref, w2_ref, b2_ref, g_ref, be_ref,
             out_ref):
    agg = jnp.concatenate([agg_ref[0], agg_ref[1]], axis=-1)
    h = x_ref[...] + agg
    h1 = jnp.dot(h, w1_ref[...], preferred_element_type=jnp.float32)
    h1 = jnp.maximum(h1 + b1_ref[...], 0.0)
    h2 = jnp.dot(h1, w2_ref[...], preferred_element_type=jnp.float32)
    h2 = h2 + b2_ref[...]
    mean = jnp.mean(h2, axis=0, keepdims=True)
    var = jnp.mean(jnp.square(h2 - mean), axis=0, keepdims=True)
    hn = (h2 - mean) * lax.rsqrt(var + 1e-5) * g_ref[...] + be_ref[...]
    out_ref[...] = jnp.maximum(hn, 0.0)


@jax.jit
def kernel(x, edge_index, W1, b1, W2, b2, bn_gamma, bn_beta):
    src = edge_index[0]
    dst = edge_index[1]
    pad = EP - E
    srcf = jnp.concatenate([src, jnp.full((pad,), N_PAD - 1, jnp.int32)])
    dstf = jnp.concatenate([dst, jnp.full((pad,), N_PAD - 1, jnp.int32)])
    eidx = jnp.stack([srcf.reshape(NBLK, NB, CH),
                      dstf.reshape(NBLK, NB, CH)], axis=1)
    zeros = jnp.zeros((ROWS_PER_TILE, DH), jnp.float32)

    agg2 = _sc_aggregate()(x, eidx, zeros)

    out = pl.pallas_call(
        _tc_body,
        out_shape=jax.ShapeDtypeStruct((N, D), jnp.float32),
    )(x, agg2, W1, b1.reshape(1, D), W2, b2.reshape(1, D),
      bn_gamma.reshape(1, D), bn_beta.reshape(1, D))
    return out


# speedup vs baseline: 7.5734x; 1.0353x over previous
import functools

import jax
import jax.numpy as jnp
from jax import lax
from jax.experimental import pallas as pl
from jax.experimental.pallas import tpu as pltpu
from jax.experimental.pallas import tpu_sc as plsc

N = 10000
E = 320000
D = 128
DH = D // 2

CH = 128
NB = 2
NTEC = 16
BLOCKS_PER_T = 80
CHUNKS_PER_T = BLOCKS_PER_T * NB
NCHUNK_PAD = CHUNKS_PER_T * NTEC
NBLK = NCHUNK_PAD // NB
EP = NCHUNK_PAD * CH
N_PAD = 10112
ROWS_PER_TILE = N_PAD // NTEC


def _sc_aggregate():
    mesh = plsc.VectorSubcoreMesh(core_axis_name="c", subcore_axis_name="s")

    @functools.partial(
        pl.kernel,
        out_type=jax.ShapeDtypeStruct((2, N, DH), jnp.float32),
        mesh=mesh,
        compiler_params=pltpu.CompilerParams(use_tc_tiling_on_sc=False),
        scratch_types=[
            pltpu.VMEM((2, 2, NB, CH), jnp.int32),
            pltpu.VMEM((2, NB, CH, DH), jnp.float32),
            pltpu.VMEM_SHARED((N_PAD, DH), jnp.float32),
            pltpu.VMEM_SHARED((N_PAD, DH), jnp.float32),
            pltpu.SemaphoreType.DMA,
            pltpu.SemaphoreType.DMA,
            pltpu.SemaphoreType.DMA,
            pltpu.SemaphoreType.DMA,
        ],
    )
    def sc_agg(x_hbm, eidx_hbm, zeros_hbm, out_hbm,
               idx_v, rows_v, agg_sh, x_sh,
               gsem0, gsem1, ssem0, ssem1):
        c = lax.axis_index("c")
        s = lax.axis_index("s")
        r0 = s * ROWS_PER_TILE
        blk0 = s * BLOCKS_PER_T
        gsem = (gsem0, gsem1)
        ssem = (ssem0, ssem1)

        def fire_gathers(p, blk):
            for j in range(NB):
                pltpu.async_copy(x_sh.at[idx_v.at[p, 0, j]],
                                 rows_v.at[p, j], gsem[p])

        def drain_gathers(p):
            for j in range(NB):
                pltpu.make_async_copy(x_sh.at[idx_v.at[p, 0, j]],
                                      rows_v.at[p, j], gsem[p]).wait()

        def fire_scatters(p):
            for j in range(NB):
                pltpu.async_copy(rows_v.at[p, j],
                                 agg_sh.at[idx_v.at[p, 1, j]], ssem[p],
                                 add=True)

        def drain_scatters(p):
            for j in range(NB):
                pltpu.make_async_copy(rows_v.at[p, j],
                                      agg_sh.at[idx_v.at[p, 1, j]],
                                      ssem[p]).wait()

        pltpu.sync_copy(zeros_hbm, agg_sh.at[pl.ds(r0, ROWS_PER_TILE)])

        @pl.when(s < NTEC - 1)
        def _stage_full():
            pltpu.sync_copy(x_hbm.at[pl.ds(r0, ROWS_PER_TILE),
                                     pl.ds(c * DH, DH)],
                            x_sh.at[pl.ds(r0, ROWS_PER_TILE)])

        @pl.when(s == NTEC - 1)
        def _stage_tail():
            real = N - (NTEC - 1) * ROWS_PER_TILE
            pltpu.sync_copy(x_hbm.at[pl.ds((NTEC - 1) * ROWS_PER_TILE, real),
                                     pl.ds(c * DH, DH)],
                            x_sh.at[pl.ds((NTEC - 1) * ROWS_PER_TILE, real)])
            pltpu.sync_copy(zeros_hbm.at[pl.ds(0, N_PAD - N)],
                            x_sh.at[pl.ds(N, N_PAD - N)])

        plsc.subcore_barrier()

        pltpu.sync_copy(eidx_hbm.at[blk0], idx_v.at[0])
        fire_gathers(0, blk0)

        def body(gi, carry):
            drain_gathers(0)

            @pl.when(gi > 0)
            def _():
                drain_scatters(1)

            pltpu.sync_copy(eidx_hbm.at[blk0 + 2 * gi + 1], idx_v.at[1])
            fire_gathers(1, blk0 + 2 * gi + 1)
            fire_scatters(0)

            drain_gathers(1)
            drain_scatters(0)

            @pl.when(gi < BLOCKS_PER_T // 2 - 1)
            def _():
                pltpu.sync_copy(eidx_hbm.at[blk0 + 2 * gi + 2], idx_v.at[0])
                fire_gathers(0, blk0 + 2 * gi + 2)

            fire_scatters(1)
            return carry

        lax.fori_loop(0, BLOCKS_PER_T // 2, body, 0)
        drain_scatters(1)
        plsc.subcore_barrier()

        @pl.when(r0 + ROWS_PER_TILE <= N)
        def _full():
            pltpu.sync_copy(agg_sh.at[pl.ds(r0, ROWS_PER_TILE)],
                            out_hbm.at[c, pl.ds(r0, ROWS_PER_TILE)])

        @pl.when(jnp.logical_and(r0 < N, r0 + ROWS_PER_TILE > N))
        def _tail():
            rb = (N // ROWS_PER_TILE) * ROWS_PER_TILE
            rem = N - rb
            pltpu.sync_copy(agg_sh.at[pl.ds(rb, rem)],
                            out_hbm.at[c, pl.ds(rb, rem)])

    return sc_agg


def _tc_body(x_ref, agg_ref, w1_ref, b1_ref, w2_ref, b2_ref, g_ref, be_ref,
             out_ref):
    agg = jnp.concatenate([agg_ref[0], agg_ref[1]], axis=-1)
    h = x_ref[...] + agg
    h1 = jnp.dot(h, w1_ref[...], preferred_element_type=jnp.float32)
    h1 = jnp.maximum(h1 + b1_ref[...], 0.0)
    h2 = jnp.dot(h1, w2_ref[...], preferred_element_type=jnp.float32)
    h2 = h2 + b2_ref[...]
    mean = jnp.mean(h2, axis=0, keepdims=True)
    var = jnp.mean(jnp.square(h2 - mean), axis=0, keepdims=True)
    hn = (h2 - mean) * lax.rsqrt(var + 1e-5) * g_ref[...] + be_ref[...]
    out_ref[...] = jnp.maximum(hn, 0.0)


@jax.jit
def kernel(x, edge_index, W1, b1, W2, b2, bn_gamma, bn_beta):
    pad = EP - E
    epad = jnp.concatenate(
        [edge_index, jnp.full((2, pad), N_PAD - 1, jnp.int32)], axis=1)
    eidx = epad.reshape(2, NBLK, NB, CH).swapaxes(0, 1)
    zeros = jnp.zeros((ROWS_PER_TILE, DH), jnp.float32)

    agg2 = _sc_aggregate()(x, eidx, zeros)

    out = pl.pallas_call(
        _tc_body,
        out_shape=jax.ShapeDtypeStruct((N, D), jnp.float32),
    )(x, agg2, W1, b1.reshape(1, D), W2, b2.reshape(1, D),
      bn_gamma.reshape(1, D), bn_beta.reshape(1, D))
    return out
